# Initial kernel scaffold; baseline (speedup 1.0000x reference)
import jax
import jax.numpy as jnp
from jax.experimental import pallas as pl


def kernel(x, edge_index, edge_attr, W1, a_src1, a_dst1, b1, W2, a_src2, a_dst2, b2, Wl, bl):
    raise NotImplementedError("write your pallas kernel here")


# speedup vs baseline: 19.5230x; 19.5230x over previous
import functools

import jax
import jax.numpy as jnp
from jax import lax
from jax.experimental import pallas as pl
from jax.experimental.pallas import tpu as pltpu
from jax.experimental.pallas import tpu_sc as plsc

N = 10000
D = 128
NPAD = 10112
DB = NPAD // D
NC = 2
NS = 16
NW = NC * NS
C = 128
CH = C // 2
ROWS_PT = NPAD // NS

NEG = -1e30


def _attn_scalars(h, a_s, a_d):
    es = jnp.dot(h, a_s, preferred_element_type=jnp.float32)
    ed = jnp.dot(h, a_d, preferred_element_type=jnp.float32)
    row = lax.broadcasted_iota(jnp.int32, (NPAD, 1), 0)
    valid = row < N
    es = jnp.where(valid, es, NEG)
    ms = jnp.max(es)
    return es, jnp.where(valid, ed, 0.0), jnp.full((1, 16), ms, jnp.float32)


_SCALAR_OUTS = [
    jax.ShapeDtypeStruct((NPAD, D), jnp.float32),
    jax.ShapeDtypeStruct((NPAD, 1), jnp.float32),
    jax.ShapeDtypeStruct((NPAD, 1), jnp.float32),
    jax.ShapeDtypeStruct((1, 16), jnp.float32),
]


def _prep_body(x_ref, w_ref, as_ref, ad_ref, h_ref, es_ref, ed_ref, ms_ref):
    h = jnp.dot(x_ref[...], w_ref[...], preferred_element_type=jnp.float32)
    h_ref[...] = h
    es_ref[...], ed_ref[...], ms_ref[...] = _attn_scalars(h, as_ref[...], ad_ref[...])


def _tc_prep(x_pad, w, a_s, a_d):
    return pl.pallas_call(_prep_body, out_shape=_SCALAR_OUTS)(x_pad, w, a_s, a_d)


def _combine(pn_ref, d0_ref, d1_ref, b_ref):
    num = pn_ref[0] + pn_ref[1]
    den = d0_ref[...] + d1_ref[...]
    z = num / (den + 1e-16) + b_ref[...]
    z = jnp.maximum(z, 0.0)
    row = lax.broadcasted_iota(jnp.int32, (NPAD, 1), 0)
    return jnp.where(row < N, z, 0.0)


def _mid_body(pn_ref, d0_ref, d1_ref, b_ref, w_ref, as_ref, ad_ref,
              h_ref, es_ref, ed_ref, ms_ref):
    z = _combine(pn_ref, d0_ref, d1_ref, b_ref)
    h = jnp.dot(z, w_ref[...], preferred_element_type=jnp.float32)
    h_ref[...] = h
    es_ref[...], ed_ref[...], ms_ref[...] = _attn_scalars(h, as_ref[...], ad_ref[...])


def _tc_mid(pn, d0, d1, b, w, a_s, a_d):
    return pl.pallas_call(_mid_body, out_shape=_SCALAR_OUTS)(pn, d0, d1, b, w, a_s, a_d)


def _fin_body(pn_ref, d0_ref, d1_ref, b_ref, wl_ref, bl_ref, out_ref):
    z = _combine(pn_ref, d0_ref, d1_ref, b_ref)
    out_ref[...] = jnp.dot(z, wl_ref[...], preferred_element_type=jnp.float32) + bl_ref[...]


def _tc_fin(pn, d0, d1, b, wl, bl):
    return pl.pallas_call(
        _fin_body,
        out_shape=jax.ShapeDtypeStruct((NPAD, 1), jnp.float32),
    )(pn, d0, d1, b, wl, bl)


def _make_sc_edge_pass(epad):
    epw = epad // NW
    nch = epw // C
    mesh = plsc.VectorSubcoreMesh(
        core_axis_name="c", subcore_axis_name="s", num_cores=NC, num_subcores=NS)

    @functools.partial(
        pl.kernel,
        out_type=(
            jax.ShapeDtypeStruct((NC, NPAD, D), jnp.float32),
            jax.ShapeDtypeStruct((NC, DB + 1, D), jnp.float32),
        ),
        mesh=mesh,
        compiler_params=pltpu.CompilerParams(needs_layout_passes=False),
        scratch_types=[
            pltpu.VMEM((NPAD,), jnp.float32),
            pltpu.VMEM((NPAD,), jnp.float32),
            pltpu.VMEM((16,), jnp.float32),
            pltpu.VMEM((C,), jnp.int32),
            pltpu.VMEM((C,), jnp.int32),
            pltpu.VMEM((C,), jnp.int32),
            pltpu.VMEM((C,), jnp.float32),
            pltpu.VMEM((C, D), jnp.float32),
            pltpu.VMEM((CH, D), jnp.float32),
            pltpu.VMEM_SHARED((NPAD, D), jnp.float32),
            pltpu.VMEM_SHARED((DB + 1, D), jnp.float32),
        ],
    )
    def sc_edge_pass(src_hbm, dst_hbm, es_hbm, ed_hbm, ms_hbm, h_hbm,
                     num_hbm, den_hbm,
                     es_v, ed_v, ms_v, src_v, dst_v, dblk_v, ex_v,
                     rows_v, exden_v, acc, den_sp):
        c = lax.axis_index("c")
        s = lax.axis_index("s")
        w = s * NC + c

        pltpu.sync_copy(es_hbm, es_v)
        pltpu.sync_copy(ed_hbm, ed_v)
        pltpu.sync_copy(ms_hbm, ms_v)

        zv = jnp.zeros((16,), jnp.float32)

        def zero_row(i, _):
            for k in range(D // 16):
                rows_v[i, pl.ds(k * 16, 16)] = zv
            return 0

        def zero_exden(i, _):
            for k in range(D // 16):
                exden_v[i, pl.ds(k * 16, 16)] = zv
            return 0

        lax.fori_loop(0, C, zero_row, 0)
        lax.fori_loop(0, CH, zero_exden, 0)
        base = s * ROWS_PT
        off = 0
        for size in (C, C, C, C, ROWS_PT - 4 * C):
            pltpu.sync_copy(rows_v.at[pl.ds(0, size)],
                            acc.at[pl.ds(pl.multiple_of(base + off, 8), size)])
            off += size

        @pl.when(s == 0)
        def _():
            pltpu.sync_copy(rows_v.at[pl.ds(0, DB + 1)], den_sp)

        plsc.subcore_barrier()

        lane = lax.iota(jnp.int32, 16)
        ms16 = ms_v[...]

        def chunk(i, _):
            eb = pl.multiple_of(w * epw + i * C, 8)
            pltpu.sync_copy(src_hbm.at[pl.ds(eb, C)], src_v)
            pltpu.sync_copy(dst_hbm.at[pl.ds(eb, C)], dst_v)
            pltpu.sync_copy(h_hbm.at[src_v], rows_v)
            for half in range(2):
                for g in range(CH // 16):
                    o = half * CH + g * 16
                    s16 = src_v[pl.ds(o, 16)]
                    d16 = dst_v[pl.ds(o, 16)]
                    es16 = plsc.load_gather(es_v, [s16])
                    ed16 = plsc.load_gather(ed_v, [d16])
                    e = es16 + ed16
                    e = jnp.where(e > 0, e, 0.2 * e)
                    t = ed16 + ms16
                    md16 = jnp.where(t > 0, t, 0.2 * t)
                    ex16 = jnp.exp(e - md16)
                    ex_v[pl.ds(o, 16)] = ex16
                    dblk_v[pl.ds(o, 16)] = lax.shift_right_logical(d16, 7)
                    plsc.store_scatter(
                        exden_v, [lane + g * 16, jnp.bitwise_and(d16, 127)], ex16)
                pltpu.sync_copy(exden_v,
                                den_sp.at[dblk_v.at[pl.ds(half * CH, CH)]],
                                add=True)
                for g in range(CH // 16):
                    o = half * CH + g * 16
                    d16 = dst_v[pl.ds(o, 16)]
                    plsc.store_scatter(
                        exden_v, [lane + g * 16, jnp.bitwise_and(d16, 127)], zv)

            def row(r, _):
                a16 = plsc.load_gather(ex_v, [lane * 0 + r])
                for k in range(D // 16):
                    rows_v[r, pl.ds(k * 16, 16)] = rows_v[r, pl.ds(k * 16, 16)] * a16
                return 0

            lax.fori_loop(0, C, row, 0)
            pltpu.sync_copy(rows_v, acc.at[dst_v], add=True)
            return 0

        lax.fori_loop(0, nch, chunk, 0)

        plsc.subcore_barrier()
        off = 0
        for size in (C, C, C, C, ROWS_PT - 4 * C):
            pltpu.sync_copy(acc.at[pl.ds(pl.multiple_of(base + off, 8), size)],
                            num_hbm.at[c, pl.ds(pl.multiple_of(base + off, 8), size)])
            off += size

        @pl.when(s == 0)
        def _():
            pltpu.sync_copy(den_sp, den_hbm.at[c])

    return sc_edge_pass


def kernel(x, edge_index, edge_attr, W1, a_src1, a_dst1, b1, W2, a_src2, a_dst2, b2, Wl, bl):
    n = x.shape[0]
    loop = jnp.arange(n, dtype=edge_index.dtype)
    src = jnp.concatenate([edge_index[0], loop])
    dst = jnp.concatenate([edge_index[1], loop])
    ep = src.shape[0]
    epad = ((ep + NW * C - 1) // (NW * C)) * (NW * C)
    src = jnp.concatenate([src, jnp.full((epad - ep,), N, jnp.int32)])
    dst = jnp.concatenate([dst, jnp.full((epad - ep,), N, jnp.int32)])

    x_pad = jnp.pad(x, ((0, NPAD - n), (0, 0)))
    sc_pass = _make_sc_edge_pass(epad)

    h1, es1, ed1, ms1 = _tc_prep(x_pad, W1, a_src1[:, None], a_dst1[:, None])
    pn1, pd1 = sc_pass(src, dst, es1[:, 0], ed1[:, 0], ms1[0], h1)
    d10 = pd1[0].reshape(-1, 1)[:NPAD]
    d11 = pd1[1].reshape(-1, 1)[:NPAD]
    h2, es2, ed2, ms2 = _tc_mid(pn1, d10, d11, b1[None, :], W2,
                                a_src2[:, None], a_dst2[:, None])
    pn2, pd2 = sc_pass(src, dst, es2[:, 0], ed2[:, 0], ms2[0], h2)
    d20 = pd2[0].reshape(-1, 1)[:NPAD]
    d21 = pd2[1].reshape(-1, 1)[:NPAD]
    out = _tc_fin(pn2, d20, d21, b2[None, :], Wl, bl[None, :])
    return out[:n]


# speedup vs baseline: 21.4704x; 1.0997x over previous
import functools

import jax
import jax.numpy as jnp
from jax import lax
from jax.experimental import pallas as pl
from jax.experimental.pallas import tpu as pltpu
from jax.experimental.pallas import tpu_sc as plsc

N = 10000
D = 128
NPAD = 10112
DB = NPAD // D
NC = 2
NS = 16
NW = NC * NS
C = 128
CH = C // 2
ROWS_PT = NPAD // NS

NEG = -1e30


def _attn_scalars(h, a_s, a_d):
    es = jnp.dot(h, a_s, preferred_element_type=jnp.float32)
    ed = jnp.dot(h, a_d, preferred_element_type=jnp.float32)
    row = lax.broadcasted_iota(jnp.int32, (NPAD, 1), 0)
    valid = row < N
    es = jnp.where(valid, es, NEG)
    ms = jnp.max(es)
    return es, jnp.where(valid, ed, 0.0), jnp.full((1, 16), ms, jnp.float32)


_SCALAR_OUTS = [
    jax.ShapeDtypeStruct((NPAD, D), jnp.float32),
    jax.ShapeDtypeStruct((NPAD, 1), jnp.float32),
    jax.ShapeDtypeStruct((NPAD, 1), jnp.float32),
    jax.ShapeDtypeStruct((1, 16), jnp.float32),
]


def _prep_body(x_ref, w_ref, as_ref, ad_ref, h_ref, es_ref, ed_ref, ms_ref):
    h = jnp.dot(x_ref[...], w_ref[...], preferred_element_type=jnp.float32)
    h_ref[...] = h
    es_ref[...], ed_ref[...], ms_ref[...] = _attn_scalars(h, as_ref[...], ad_ref[...])


def _tc_prep(x_pad, w, a_s, a_d):
    return pl.pallas_call(_prep_body, out_shape=_SCALAR_OUTS)(x_pad, w, a_s, a_d)


def _combine(pn_ref, d0_ref, d1_ref, b_ref):
    num = pn_ref[0] + pn_ref[1]
    den = d0_ref[...] + d1_ref[...]
    z = num / (den + 1e-16) + b_ref[...]
    z = jnp.maximum(z, 0.0)
    row = lax.broadcasted_iota(jnp.int32, (NPAD, 1), 0)
    return jnp.where(row < N, z, 0.0)


def _mid_body(pn_ref, d0_ref, d1_ref, b_ref, w_ref, as_ref, ad_ref,
              h_ref, es_ref, ed_ref, ms_ref):
    z = _combine(pn_ref, d0_ref, d1_ref, b_ref)
    h = jnp.dot(z, w_ref[...], preferred_element_type=jnp.float32)
    h_ref[...] = h
    es_ref[...], ed_ref[...], ms_ref[...] = _attn_scalars(h, as_ref[...], ad_ref[...])


def _tc_mid(pn, d0, d1, b, w, a_s, a_d):
    return pl.pallas_call(_mid_body, out_shape=_SCALAR_OUTS)(pn, d0, d1, b, w, a_s, a_d)


def _fin_body(pn_ref, d0_ref, d1_ref, b_ref, wl_ref, bl_ref, out_ref):
    z = _combine(pn_ref, d0_ref, d1_ref, b_ref)
    out_ref[...] = jnp.dot(z, wl_ref[...], preferred_element_type=jnp.float32) + bl_ref[...]


def _tc_fin(pn, d0, d1, b, wl, bl):
    return pl.pallas_call(
        _fin_body,
        out_shape=jax.ShapeDtypeStruct((NPAD, 1), jnp.float32),
    )(pn, d0, d1, b, wl, bl)


def _make_sc_edge_pass(epad):
    epw = epad // NW
    nch = epw // C
    mesh = plsc.VectorSubcoreMesh(
        core_axis_name="c", subcore_axis_name="s", num_cores=NC, num_subcores=NS)

    @functools.partial(
        pl.kernel,
        out_type=(
            jax.ShapeDtypeStruct((NC, NPAD, D), jnp.float32),
            jax.ShapeDtypeStruct((NC, DB + 1, D), jnp.float32),
        ),
        mesh=mesh,
        compiler_params=pltpu.CompilerParams(needs_layout_passes=False),
        scratch_types=[
            pltpu.VMEM((NPAD,), jnp.float32),
            pltpu.VMEM((NPAD,), jnp.float32),
            pltpu.VMEM((16,), jnp.float32),
            pltpu.VMEM((CH,), jnp.int32),
            pltpu.VMEM((CH,), jnp.int32),
            pltpu.VMEM((CH,), jnp.int32),
            pltpu.VMEM((CH,), jnp.int32),
            pltpu.VMEM((CH,), jnp.int32),
            pltpu.VMEM((CH,), jnp.int32),
            pltpu.VMEM((C,), jnp.float32),
            pltpu.VMEM((C, D), jnp.float32),
            pltpu.VMEM((CH, D), jnp.float32),
            pltpu.VMEM_SHARED((NPAD, D), jnp.float32),
            pltpu.VMEM_SHARED((DB + 1, D), jnp.float32),
            pltpu.SemaphoreType.DMA,
            pltpu.SemaphoreType.DMA,
            pltpu.SemaphoreType.DMA,
            pltpu.SemaphoreType.DMA,
        ],
    )
    def sc_edge_pass(src_hbm, dst_hbm, es_hbm, ed_hbm, ms_hbm, h_hbm,
                     num_hbm, den_hbm,
                     es_v, ed_v, ms_v, srcA_v, srcB_v, dstA_v, dstB_v,
                     dblkA_v, dblkB_v, ex_v, rows_v, exden_v, acc, den_sp,
                     gsA, gsB, ssA, ssB):
        c = lax.axis_index("c")
        s = lax.axis_index("s")
        w = s * NC + c

        pltpu.sync_copy(es_hbm, es_v)
        pltpu.sync_copy(ed_hbm, ed_v)
        pltpu.sync_copy(ms_hbm, ms_v)

        zv = jnp.zeros((16,), jnp.float32)

        def zero_row(i, _):
            for k in range(D // 16):
                rows_v[i, pl.ds(k * 16, 16)] = zv
            return 0

        def zero_exden(i, _):
            for k in range(D // 16):
                exden_v[i, pl.ds(k * 16, 16)] = zv
            return 0

        lax.fori_loop(0, C, zero_row, 0)
        lax.fori_loop(0, CH, zero_exden, 0)
        base = s * ROWS_PT
        off = 0
        for size in (C, C, C, C, ROWS_PT - 4 * C):
            pltpu.sync_copy(rows_v.at[pl.ds(0, size)],
                            acc.at[pl.ds(pl.multiple_of(base + off, 8), size)])
            off += size

        @pl.when(s == 0)
        def _():
            pltpu.sync_copy(rows_v.at[pl.ds(0, DB + 1)], den_sp)

        plsc.subcore_barrier()

        lane = lax.iota(jnp.int32, 16)
        ms16 = ms_v[...]
        halves = ((0, srcA_v, dstA_v, dblkA_v, gsA, ssA),
                  (CH, srcB_v, dstB_v, dblkB_v, gsB, ssB))

        def chunk(i, _):
            eb = pl.multiple_of(w * epw + i * C, 8)

            @pl.when(i > 0)
            def _():
                for o, _s, d_ref, _b, _g, ssem in halves:
                    pltpu.make_async_copy(
                        rows_v.at[pl.ds(o, CH)], acc.at[d_ref], ssem).wait()

            for o, s_ref, d_ref, _b, _g, _s2 in halves:
                pltpu.sync_copy(src_hbm.at[pl.ds(eb + o, CH)], s_ref)
                pltpu.sync_copy(dst_hbm.at[pl.ds(eb + o, CH)], d_ref)
            gA = pltpu.async_copy(h_hbm.at[srcA_v], rows_v.at[pl.ds(0, CH)], gsA)
            gB = pltpu.async_copy(h_hbm.at[srcB_v], rows_v.at[pl.ds(CH, CH)], gsB)

            for o, s_ref, d_ref, blk_ref, _g, _s2 in halves:
                for g in range(CH // 16):
                    s16 = s_ref[pl.ds(g * 16, 16)]
                    d16 = d_ref[pl.ds(g * 16, 16)]
                    es16 = plsc.load_gather(es_v, [s16])
                    ed16 = plsc.load_gather(ed_v, [d16])
                    e = es16 + ed16
                    e = jnp.where(e > 0, e, 0.2 * e)
                    t = ed16 + ms16
                    md16 = jnp.where(t > 0, t, 0.2 * t)
                    ex16 = jnp.exp(e - md16)
                    ex_v[pl.ds(o + g * 16, 16)] = ex16
                    blk_ref[pl.ds(g * 16, 16)] = lax.shift_right_logical(d16, 7)
                    plsc.store_scatter(
                        exden_v, [lane + g * 16, jnp.bitwise_and(d16, 127)], ex16)
                pltpu.sync_copy(exden_v, den_sp.at[blk_ref], add=True)
                for g in range(CH // 16):
                    d16 = d_ref[pl.ds(g * 16, 16)]
                    plsc.store_scatter(
                        exden_v, [lane + g * 16, jnp.bitwise_and(d16, 127)], zv)

            for (o, _s, d_ref, _b, _g2, ssem), gdesc in zip(halves, (gA, gB)):
                gdesc.wait()

                def row(j, _, o=o):
                    for u in range(2):
                        r = o + 2 * j + u
                        a16 = plsc.load_gather(ex_v, [lane * 0 + r])
                        for k in range(D // 16):
                            rows_v[r, pl.ds(k * 16, 16)] = (
                                rows_v[r, pl.ds(k * 16, 16)] * a16)
                    return 0

                lax.fori_loop(0, CH // 2, row, 0)
                pltpu.async_copy(
                    rows_v.at[pl.ds(o, CH)], acc.at[d_ref], ssem, add=True)
            return 0

        lax.fori_loop(0, nch, chunk, 0)

        for o, _s, d_ref, _b, _g, ssem in halves:
            pltpu.make_async_copy(
                rows_v.at[pl.ds(o, CH)], acc.at[d_ref], ssem).wait()

        plsc.subcore_barrier()
        off = 0
        for size in (C, C, C, C, ROWS_PT - 4 * C):
            pltpu.sync_copy(acc.at[pl.ds(pl.multiple_of(base + off, 8), size)],
                            num_hbm.at[c, pl.ds(pl.multiple_of(base + off, 8), size)])
            off += size

        @pl.when(s == 0)
        def _():
            pltpu.sync_copy(den_sp, den_hbm.at[c])

    return sc_edge_pass


def kernel(x, edge_index, edge_attr, W1, a_src1, a_dst1, b1, W2, a_src2, a_dst2, b2, Wl, bl):
    n = x.shape[0]
    loop = jnp.arange(n, dtype=edge_index.dtype)
    src = jnp.concatenate([edge_index[0], loop])
    dst = jnp.concatenate([edge_index[1], loop])
    ep = src.shape[0]
    epad = ((ep + NW * C - 1) // (NW * C)) * (NW * C)
    src = jnp.concatenate([src, jnp.full((epad - ep,), N, jnp.int32)])
    dst = jnp.concatenate([dst, jnp.full((epad - ep,), N, jnp.int32)])

    x_pad = jnp.pad(x, ((0, NPAD - n), (0, 0)))
    sc_pass = _make_sc_edge_pass(epad)

    h1, es1, ed1, ms1 = _tc_prep(x_pad, W1, a_src1[:, None], a_dst1[:, None])
    pn1, pd1 = sc_pass(src, dst, es1[:, 0], ed1[:, 0], ms1[0], h1)
    d10 = pd1[0].reshape(-1, 1)[:NPAD]
    d11 = pd1[1].reshape(-1, 1)[:NPAD]
    h2, es2, ed2, ms2 = _tc_mid(pn1, d10, d11, b1[None, :], W2,
                                a_src2[:, None], a_dst2[:, None])
    pn2, pd2 = sc_pass(src, dst, es2[:, 0], ed2[:, 0], ms2[0], h2)
    d20 = pd2[0].reshape(-1, 1)[:NPAD]
    d21 = pd2[1].reshape(-1, 1)[:NPAD]
    out = _tc_fin(pn2, d20, d21, b2[None, :], Wl, bl[None, :])
    return out[:n]
